# baseline (speedup 1.0000x reference)
import functools

import jax
import jax.numpy as jnp
from jax import lax
from jax.experimental import pallas as pl
from jax.experimental.pallas import tpu as pltpu
from jax.experimental.pallas import tpu_sc as plsc

_N = 10000
_D = 128
_L = 16
_NSUB = 16
_K = 128
_G = 16
_NBLK = 10
_NCHUNK = _G * _NBLK
_EPW = _NCHUNK * _K
_EPAD = _EPW * _NSUB
_RPS = 640
_ACC_ROWS = _RPS * _NSUB
_RBLK = 1000


def _make_prop():
    mesh = plsc.VectorSubcoreMesh(core_axis_name="c", subcore_axis_name="s")

    @functools.partial(
        pl.kernel,
        mesh=mesh,
        out_type=[
            jax.ShapeDtypeStruct((_ACC_ROWS, _D), jnp.float32),
            jax.ShapeDtypeStruct((_ACC_ROWS, _D), jnp.float32),
        ],
        scratch_types=[
            pltpu.VMEM((_G, _K), jnp.int32),
            pltpu.VMEM((_G, _K), jnp.int32),
            pltpu.VMEM((_G, _K), jnp.float32),
            pltpu.VMEM((2, _K, _D), jnp.float32),
            pltpu.VMEM_SHARED((_ACC_ROWS, _D), jnp.float32),
            pltpu.SemaphoreType.DMA,
            pltpu.SemaphoreType.DMA,
            pltpu.SemaphoreType.DMA,
            pltpu.SemaphoreType.DMA,
        ],
    )
    def prop(yin_hbm, yout_hbm, src_hbm, dst_hbm, ew_hbm, z_hbm,
             pin_hbm, pout_hbm,
             srcb, dstb, ewb, rowsb, acc, sem0, sem1, ssem0, ssem1):
        cid = lax.axis_index("c")
        sid = lax.axis_index("s")
        rs = sid * _RPS
        pltpu.sync_copy(z_hbm, acc.at[pl.ds(rs, _RPS)])
        plsc.subcore_barrier()
        sems = (sem0, sem1)
        ssems = (ssem0, ssem1)

        def run(table_hbm):
            def gather(c, b):
                return pltpu.make_async_copy(table_hbm.at[srcb.at[c]],
                                             rowsb.at[b], sems[b])

            def scat_start(c, b):
                pltpu.async_copy(rowsb.at[b], acc.at[dstb.at[c]], ssems[b],
                                 add=True)

            def scat_wait(c, b):
                pltpu.make_async_copy(rowsb.at[b], acc.at[dstb.at[c]],
                                      ssems[b]).wait()

            def scale(c, b):
                def body(g, cr):
                    wv16 = ewb[c, pl.ds(g * _L, _L)]
                    for t in range(_L):
                        wv = jnp.full((_L,), wv16[t], dtype=jnp.float32)
                        e = g * _L + t
                        for j in range(_D // _L):
                            sl = pl.ds(j * _L, _L)
                            rowsb[b, e, sl] = rowsb[b, e, sl] * wv
                    return cr
                lax.fori_loop(0, _K // _L, body, 0)

            def blk_body(blk, carry):
                g0 = blk * _G
                pltpu.sync_copy(src_hbm.at[sid, pl.ds(g0, _G)], srcb)
                pltpu.sync_copy(dst_hbm.at[sid, pl.ds(g0, _G)], dstb)
                pltpu.sync_copy(ew_hbm.at[sid, pl.ds(g0, _G)], ewb)
                gather(0, 0).start()

                def pair_body(p, carry2):
                    c0 = p * 2
                    @pl.when(p > 0)
                    def _():
                        scat_wait(c0 - 1, 1)

                    gather(c0 + 1, 1).start()
                    gather(c0, 0).wait()
                    scale(c0, 0)
                    scat_start(c0, 0)
                    gather(c0 + 1, 1).wait()
                    scale(c0 + 1, 1)
                    scat_start(c0 + 1, 1)
                    scat_wait(c0, 0)

                    @pl.when(c0 + 2 < _G)
                    def _():
                        gather(c0 + 2, 0).start()

                    return carry2

                lax.fori_loop(0, _G // 2, pair_body, 0)
                scat_wait(_G - 1, 1)
                return carry

            lax.fori_loop(0, _NBLK, blk_body, 0)

        @pl.when(cid == 0)
        def _():
            run(yin_hbm)

        @pl.when(cid == 1)
        def _():
            run(yout_hbm)

        plsc.subcore_barrier()

        @pl.when(cid == 0)
        def _():
            pltpu.sync_copy(acc.at[pl.ds(rs, _RPS)],
                            pin_hbm.at[pl.ds(rs, _RPS)])

        @pl.when(cid == 1)
        def _():
            pltpu.sync_copy(acc.at[pl.ds(rs, _RPS)],
                            pout_hbm.at[pl.ds(rs, _RPS)])

    return prop


def _tc_in(x, pe_row, wmi_t, wmo_t, ws_t):
    grid = (_N // _RBLK,)

    def body(x_ref, pe_ref, wmi_ref, wmo_ref, ws_ref, h_ref, yin_ref, yout_ref):
        h = x_ref[:] + pe_ref[:]
        h_ref[:] = h
        ds = jnp.dot(h, ws_ref[:], preferred_element_type=jnp.float32)
        yin_ref[:] = jnp.dot(h, wmi_ref[:],
                             preferred_element_type=jnp.float32) + ds
        yout_ref[:] = jnp.dot(h, wmo_ref[:],
                              preferred_element_type=jnp.float32) + ds

    blk = pl.BlockSpec((_RBLK, _D), lambda i: (i, 0))
    one = pl.BlockSpec((1, _D), lambda i: (0, 0))
    wblk = pl.BlockSpec((_D, _D), lambda i: (0, 0))
    return pl.pallas_call(
        body,
        grid=grid,
        in_specs=[blk, one, wblk, wblk, wblk],
        out_specs=[blk, blk, blk],
        out_shape=[jax.ShapeDtypeStruct((_N, _D), jnp.float32)] * 3,
    )(x, pe_row, wmi_t, wmo_t, ws_t)


def _tc_mid(pin, pout, h_prev, cin, cout, b_in, b_out, wmi_t, wmo_t, ws_t):
    grid = (_N // _RBLK,)

    def body(pin_ref, pout_ref, h_ref, cin_ref, cout_ref, bin_ref, bout_ref,
             wmi_ref, wmo_ref, ws_ref, hn_ref, yin_ref, yout_ref):
        ic = pin_ref[:] + bin_ref[:]
        oc = pout_ref[:] + bout_ref[:]
        h = jnp.tanh(cin_ref[:] * ic + cout_ref[:] * oc + h_ref[:])
        hn_ref[:] = h
        ds = jnp.dot(h, ws_ref[:], preferred_element_type=jnp.float32)
        yin_ref[:] = jnp.dot(h, wmi_ref[:],
                             preferred_element_type=jnp.float32) + ds
        yout_ref[:] = jnp.dot(h, wmo_ref[:],
                              preferred_element_type=jnp.float32) + ds

    blk = pl.BlockSpec((_RBLK, _D), lambda i: (i, 0))
    cblk = pl.BlockSpec((_RBLK, 1), lambda i: (i, 0))
    one = pl.BlockSpec((1, _D), lambda i: (0, 0))
    wblk = pl.BlockSpec((_D, _D), lambda i: (0, 0))
    return pl.pallas_call(
        body,
        grid=grid,
        in_specs=[blk, blk, blk, cblk, cblk, one, one, wblk, wblk, wblk],
        out_specs=[blk, blk, blk],
        out_shape=[jax.ShapeDtypeStruct((_N, _D), jnp.float32)] * 3,
    )(pin, pout, h_prev, cin, cout, b_in, b_out, wmi_t, wmo_t, ws_t)


def _tc_out(pin, pout, h_prev, cin, cout, b_in, b_out, wd_t, bd_pad):
    grid = (_N // _RBLK,)

    def body(pin_ref, pout_ref, h_ref, cin_ref, cout_ref, bin_ref, bout_ref,
             wd_ref, bd_ref, logp_ref, emb_ref):
        ic = pin_ref[:] + bin_ref[:]
        oc = pout_ref[:] + bout_ref[:]
        h3 = jnp.tanh(cin_ref[:] * ic + cout_ref[:] * oc + h_ref[:])
        logits = jnp.dot(h3, wd_ref[:],
                         preferred_element_type=jnp.float32) + bd_ref[:]
        m = jnp.max(logits, axis=-1, keepdims=True)
        lse = m + jnp.log(jnp.sum(jnp.exp(logits - m), axis=-1, keepdims=True))
        logp_ref[:] = logits - lse
        nrm = jnp.sqrt(jnp.sum(h3 * h3, axis=-1, keepdims=True))
        emb_ref[:] = h3 / (nrm + 1e-12)

    blk = pl.BlockSpec((_RBLK, _D), lambda i: (i, 0))
    cblk = pl.BlockSpec((_RBLK, 1), lambda i: (i, 0))
    one = pl.BlockSpec((1, _D), lambda i: (0, 0))
    wblk = pl.BlockSpec((_D, _D), lambda i: (0, 0))
    return pl.pallas_call(
        body,
        grid=grid,
        in_specs=[blk, blk, blk, cblk, cblk, one, one, wblk, one],
        out_specs=[blk, blk],
        out_shape=[jax.ShapeDtypeStruct((_N, _D), jnp.float32)] * 2,
    )(pin, pout, h_prev, cin, cout, b_in, b_out, wd_t, bd_pad)


def kernel(x, edge_index, edge_weight, pe,
           W1_main_in, W1_main_out, W1_skip, b1_main_in, b1_main_out,
           b1_skip_in, b1_skip_out, C1_in, C1_out,
           W2_main_in, W2_main_out, W2_skip, b2_main_in, b2_main_out,
           b2_skip_in, b2_skip_out, C2_in, C2_out,
           W3_main_in, W3_main_out, W3_skip, b3_main_in, b3_main_out,
           b3_skip_in, b3_skip_out, C3_in, C3_out,
           Wd, bd):
    src, dst = edge_index[0], edge_index[1]

    pad = _EPAD - src.shape[0]
    ar = jnp.arange(pad, dtype=jnp.int32)
    src_p = jnp.concatenate([src, ar % _N]).reshape(_NSUB, _NCHUNK, _K)
    dst_p = jnp.concatenate([dst, _N + (ar % _NSUB)]).reshape(
        _NSUB, _NCHUNK, _K)
    ew_p = jnp.concatenate([edge_weight,
                            jnp.zeros((pad,), jnp.float32)]).reshape(
        _NSUB, _NCHUNK, _K)
    zrows = jnp.zeros((_RPS, _D), jnp.float32)

    def mk(bmi, bmo, bsi, bso):
        return ((bmi + bsi).reshape(1, _D), (bmo + bso).reshape(1, _D))

    bi1, bo1 = mk(b1_main_in, b1_main_out, b1_skip_in, b1_skip_out)
    bi2, bo2 = mk(b2_main_in, b2_main_out, b2_skip_in, b2_skip_out)
    bi3, bo3 = mk(b3_main_in, b3_main_out, b3_skip_in, b3_skip_out)

    pe_row = pe[:4].reshape(1, _D)
    wd_t = jnp.zeros((_D, _D), jnp.float32).at[:, :25].set(Wd.T)
    bd_pad = jnp.full((1, _D), -1e30, jnp.float32).at[0, :25].set(bd)

    prop = _make_prop()

    x_pe, y1i, y1o = _tc_in(x, pe_row, W1_main_in.T, W1_main_out.T, W1_skip.T)
    p1i, p1o = prop(y1i, y1o, src_p, dst_p, ew_p, zrows)
    h1, y2i, y2o = _tc_mid(p1i[:_N], p1o[:_N], x_pe, C1_in, C1_out,
                           bi1, bo1, W2_main_in.T, W2_main_out.T, W2_skip.T)
    p2i, p2o = prop(y2i, y2o, src_p, dst_p, ew_p, zrows)
    h2, y3i, y3o = _tc_mid(p2i[:_N], p2o[:_N], h1, C2_in, C2_out,
                           bi2, bo2, W3_main_in.T, W3_main_out.T, W3_skip.T)
    p3i, p3o = prop(y3i, y3o, src_p, dst_p, ew_p, zrows)
    logp_pad, emb = _tc_out(p3i[:_N], p3o[:_N], h2, C3_in, C3_out,
                            bi3, bo3, wd_t, bd_pad)
    return (logp_pad[:, :25], emb)


# speedup vs baseline: 1.1067x; 1.1067x over previous
import functools

import jax
import jax.numpy as jnp
from jax import lax
from jax.experimental import pallas as pl
from jax.experimental.pallas import tpu as pltpu
from jax.experimental.pallas import tpu_sc as plsc

_N = 10000
_D = 128
_L = 16
_NSUB = 16
_K = 112
_G = 16
_NBLK = 12
_NCHUNK = _G * _NBLK
_EPW = _NCHUNK * _K
_EPAD = _EPW * _NSUB
_RPS = 632
_ACC_ROWS = _RPS * _NSUB
_RBLK = 1000


def _make_prop():
    mesh = plsc.VectorSubcoreMesh(core_axis_name="c", subcore_axis_name="s")

    @functools.partial(
        pl.kernel,
        mesh=mesh,
        out_type=[
            jax.ShapeDtypeStruct((_ACC_ROWS, _D), jnp.float32),
            jax.ShapeDtypeStruct((_ACC_ROWS, _D), jnp.float32),
        ],
        scratch_types=[
            pltpu.VMEM((_G, _K), jnp.int32),
            pltpu.VMEM((_G, _K), jnp.int32),
            pltpu.VMEM((_G, _K), jnp.float32),
            pltpu.VMEM((3, _K, _D), jnp.float32),
            pltpu.VMEM_SHARED((_ACC_ROWS, _D), jnp.float32),
            pltpu.SemaphoreType.DMA,
            pltpu.SemaphoreType.DMA,
            pltpu.SemaphoreType.DMA,
            pltpu.SemaphoreType.DMA,
            pltpu.SemaphoreType.DMA,
            pltpu.SemaphoreType.DMA,
        ],
    )
    def prop(yin_hbm, yout_hbm, src_hbm, dst_hbm, ew_hbm, z_hbm,
             pin_hbm, pout_hbm,
             srcb, dstb, ewb, rowsb, acc,
             sem0, sem1, sem2, ssem0, ssem1, ssem2):
        cid = lax.axis_index("c")
        sid = lax.axis_index("s")
        rs = sid * _RPS
        pltpu.sync_copy(z_hbm, acc.at[pl.ds(rs, _RPS)])
        plsc.subcore_barrier()
        sems = (sem0, sem1, sem2)
        ssems = (ssem0, ssem1, ssem2)

        def run(table_hbm):
            def gather(c, b):
                return pltpu.make_async_copy(table_hbm.at[srcb.at[c]],
                                             rowsb.at[b], sems[b])

            def scat_start(c, b):
                pltpu.async_copy(rowsb.at[b], acc.at[dstb.at[c]], ssems[b],
                                 add=True)

            def scat_wait(c, b):
                pltpu.make_async_copy(rowsb.at[b], acc.at[dstb.at[c]],
                                      ssems[b]).wait()

            def scale(c, b):
                def body(g, cr):
                    wv16 = ewb[c, pl.ds(g * _L, _L)]
                    for t in range(_L):
                        wv = jnp.full((_L,), wv16[t], dtype=jnp.float32)
                        e = g * _L + t
                        for j in range(_D // _L):
                            sl = pl.ds(j * _L, _L)
                            rowsb[b, e, sl] = rowsb[b, e, sl] * wv
                    return cr
                lax.fori_loop(0, _K // _L, body, 0)

            def blk_body(blk, carry):
                g0 = blk * _G
                pltpu.sync_copy(src_hbm.at[sid, pl.ds(g0, _G)], srcb)
                pltpu.sync_copy(dst_hbm.at[sid, pl.ds(g0, _G)], dstb)
                pltpu.sync_copy(ew_hbm.at[sid, pl.ds(g0, _G)], ewb)
                gather(0, 0).start()
                gather(1, 1).start()
                for c in range(_G):
                    b = c % 3
                    gather(c, b).wait()
                    scale(c, b)
                    scat_start(c, b)
                    nxt = c + 2
                    if nxt < _G:
                        nb = nxt % 3
                        if nxt >= 3:
                            scat_wait(nxt - 3, nb)
                        gather(nxt, nb).start()
                for c in range(_G - 3, _G):
                    scat_wait(c, c % 3)
                return carry

            lax.fori_loop(0, _NBLK, blk_body, 0)

        @pl.when(cid == 0)
        def _():
            run(yin_hbm)

        @pl.when(cid == 1)
        def _():
            run(yout_hbm)

        plsc.subcore_barrier()

        @pl.when(cid == 0)
        def _():
            pltpu.sync_copy(acc.at[pl.ds(rs, _RPS)],
                            pin_hbm.at[pl.ds(rs, _RPS)])

        @pl.when(cid == 1)
        def _():
            pltpu.sync_copy(acc.at[pl.ds(rs, _RPS)],
                            pout_hbm.at[pl.ds(rs, _RPS)])

    return prop


def _tc_in(x, pe_row, wmi_t, wmo_t, ws_t):
    grid = (_N // _RBLK,)

    def body(x_ref, pe_ref, wmi_ref, wmo_ref, ws_ref, h_ref, yin_ref, yout_ref):
        h = x_ref[:] + pe_ref[:]
        h_ref[:] = h
        ds = jnp.dot(h, ws_ref[:], preferred_element_type=jnp.float32)
        yin_ref[:] = jnp.dot(h, wmi_ref[:],
                             preferred_element_type=jnp.float32) + ds
        yout_ref[:] = jnp.dot(h, wmo_ref[:],
                              preferred_element_type=jnp.float32) + ds

    blk = pl.BlockSpec((_RBLK, _D), lambda i: (i, 0))
    one = pl.BlockSpec((1, _D), lambda i: (0, 0))
    wblk = pl.BlockSpec((_D, _D), lambda i: (0, 0))
    return pl.pallas_call(
        body,
        grid=grid,
        in_specs=[blk, one, wblk, wblk, wblk],
        out_specs=[blk, blk, blk],
        out_shape=[jax.ShapeDtypeStruct((_N, _D), jnp.float32)] * 3,
    )(x, pe_row, wmi_t, wmo_t, ws_t)


def _tc_mid(pin, pout, h_prev, cin, cout, b_in, b_out, wmi_t, wmo_t, ws_t):
    grid = (_N // _RBLK,)

    def body(pin_ref, pout_ref, h_ref, cin_ref, cout_ref, bin_ref, bout_ref,
             wmi_ref, wmo_ref, ws_ref, hn_ref, yin_ref, yout_ref):
        ic = pin_ref[:] + bin_ref[:]
        oc = pout_ref[:] + bout_ref[:]
        h = jnp.tanh(cin_ref[:] * ic + cout_ref[:] * oc + h_ref[:])
        hn_ref[:] = h
        ds = jnp.dot(h, ws_ref[:], preferred_element_type=jnp.float32)
        yin_ref[:] = jnp.dot(h, wmi_ref[:],
                             preferred_element_type=jnp.float32) + ds
        yout_ref[:] = jnp.dot(h, wmo_ref[:],
                              preferred_element_type=jnp.float32) + ds

    blk = pl.BlockSpec((_RBLK, _D), lambda i: (i, 0))
    cblk = pl.BlockSpec((_RBLK, 1), lambda i: (i, 0))
    one = pl.BlockSpec((1, _D), lambda i: (0, 0))
    wblk = pl.BlockSpec((_D, _D), lambda i: (0, 0))
    return pl.pallas_call(
        body,
        grid=grid,
        in_specs=[blk, blk, blk, cblk, cblk, one, one, wblk, wblk, wblk],
        out_specs=[blk, blk, blk],
        out_shape=[jax.ShapeDtypeStruct((_N, _D), jnp.float32)] * 3,
    )(pin, pout, h_prev, cin, cout, b_in, b_out, wmi_t, wmo_t, ws_t)


def _tc_out(pin, pout, h_prev, cin, cout, b_in, b_out, wd_t, bd_pad):
    grid = (_N // _RBLK,)

    def body(pin_ref, pout_ref, h_ref, cin_ref, cout_ref, bin_ref, bout_ref,
             wd_ref, bd_ref, logp_ref, emb_ref):
        ic = pin_ref[:] + bin_ref[:]
        oc = pout_ref[:] + bout_ref[:]
        h3 = jnp.tanh(cin_ref[:] * ic + cout_ref[:] * oc + h_ref[:])
        logits = jnp.dot(h3, wd_ref[:],
                         preferred_element_type=jnp.float32) + bd_ref[:]
        m = jnp.max(logits, axis=-1, keepdims=True)
        lse = m + jnp.log(jnp.sum(jnp.exp(logits - m), axis=-1, keepdims=True))
        logp_ref[:] = logits - lse
        nrm = jnp.sqrt(jnp.sum(h3 * h3, axis=-1, keepdims=True))
        emb_ref[:] = h3 / (nrm + 1e-12)

    blk = pl.BlockSpec((_RBLK, _D), lambda i: (i, 0))
    cblk = pl.BlockSpec((_RBLK, 1), lambda i: (i, 0))
    one = pl.BlockSpec((1, _D), lambda i: (0, 0))
    wblk = pl.BlockSpec((_D, _D), lambda i: (0, 0))
    return pl.pallas_call(
        body,
        grid=grid,
        in_specs=[blk, blk, blk, cblk, cblk, one, one, wblk, one],
        out_specs=[blk, blk],
        out_shape=[jax.ShapeDtypeStruct((_N, _D), jnp.float32)] * 2,
    )(pin, pout, h_prev, cin, cout, b_in, b_out, wd_t, bd_pad)


def kernel(x, edge_index, edge_weight, pe,
           W1_main_in, W1_main_out, W1_skip, b1_main_in, b1_main_out,
           b1_skip_in, b1_skip_out, C1_in, C1_out,
           W2_main_in, W2_main_out, W2_skip, b2_main_in, b2_main_out,
           b2_skip_in, b2_skip_out, C2_in, C2_out,
           W3_main_in, W3_main_out, W3_skip, b3_main_in, b3_main_out,
           b3_skip_in, b3_skip_out, C3_in, C3_out,
           Wd, bd):
    src, dst = edge_index[0], edge_index[1]

    pad = _EPAD - src.shape[0]
    ar = jnp.arange(pad, dtype=jnp.int32)
    src_p = jnp.concatenate([src, ar % _N]).reshape(_NSUB, _NCHUNK, _K)
    dst_p = jnp.concatenate([dst, _N + (ar % _NSUB)]).reshape(
        _NSUB, _NCHUNK, _K)
    ew_p = jnp.concatenate([edge_weight,
                            jnp.zeros((pad,), jnp.float32)]).reshape(
        _NSUB, _NCHUNK, _K)
    zrows = jnp.zeros((_RPS, _D), jnp.float32)

    def mk(bmi, bmo, bsi, bso):
        return ((bmi + bsi).reshape(1, _D), (bmo + bso).reshape(1, _D))

    bi1, bo1 = mk(b1_main_in, b1_main_out, b1_skip_in, b1_skip_out)
    bi2, bo2 = mk(b2_main_in, b2_main_out, b2_skip_in, b2_skip_out)
    bi3, bo3 = mk(b3_main_in, b3_main_out, b3_skip_in, b3_skip_out)

    pe_row = pe[:4].reshape(1, _D)
    wd_t = jnp.zeros((_D, _D), jnp.float32).at[:, :25].set(Wd.T)
    bd_pad = jnp.full((1, _D), -1e30, jnp.float32).at[0, :25].set(bd)

    prop = _make_prop()

    x_pe, y1i, y1o = _tc_in(x, pe_row, W1_main_in.T, W1_main_out.T, W1_skip.T)
    p1i, p1o = prop(y1i, y1o, src_p, dst_p, ew_p, zrows)
    h1, y2i, y2o = _tc_mid(p1i[:_N], p1o[:_N], x_pe, C1_in, C1_out,
                           bi1, bo1, W2_main_in.T, W2_main_out.T, W2_skip.T)
    p2i, p2o = prop(y2i, y2o, src_p, dst_p, ew_p, zrows)
    h2, y3i, y3o = _tc_mid(p2i[:_N], p2o[:_N], h1, C2_in, C2_out,
                           bi2, bo2, W3_main_in.T, W3_main_out.T, W3_skip.T)
    p3i, p3o = prop(y3i, y3o, src_p, dst_p, ew_p, zrows)
    logp_pad, emb = _tc_out(p3i[:_N], p3o[:_N], h2, C3_in, C3_out,
                            bi3, bo3, wd_t, bd_pad)
    return (logp_pad[:, :25], emb)
